# baseline (speedup 1.0000x reference)
import jax
import jax.numpy as jnp
from jax.experimental import pallas as pl
from jax.experimental.pallas import tpu as pltpu

_H = 32
_W = 32
_RH = 16


def _add_body(x_ref, row_ref, col_ref, o_ref, pe_ref):
    j = pl.program_id(1)

    @pl.when((pl.program_id(0) == 0) & (j == 0))
    def _():
        pe_ref[...] = row_ref[...][:, None, :] + col_ref[...][None, :, :]

    o_ref[...] = x_ref[...] + pe_ref[pl.ds(j * _RH, _RH)][None]


def kernel(x, pos_row, pos_col):
    B, SEQ, D = x.shape
    x4 = x.reshape(B, _H, _W, D)
    out = pl.pallas_call(
        _add_body,
        grid=(B, _H // _RH),
        in_specs=[
            pl.BlockSpec((1, _RH, _W, D), lambda b, j: (b, j, 0, 0)),
            pl.BlockSpec((_H, D), lambda b, j: (0, 0)),
            pl.BlockSpec((_W, D), lambda b, j: (0, 0)),
        ],
        out_specs=pl.BlockSpec((1, _RH, _W, D), lambda b, j: (b, j, 0, 0)),
        out_shape=jax.ShapeDtypeStruct((B, _H, _W, D), x.dtype),
        scratch_shapes=[pltpu.VMEM((_H, _W, D), x.dtype)],
    )(x4, pos_row, pos_col)
    return out.reshape(B, SEQ, D)


# speedup vs baseline: 1.2481x; 1.2481x over previous
import jax
import jax.numpy as jnp
from jax.experimental import pallas as pl
from jax.experimental.pallas import tpu as pltpu

_H = 32
_W = 32
_NB = 2


def _add_body(x_ref, row_ref, col_ref, o_ref, pe_ref):
    @pl.when(pl.program_id(0) == 0)
    def _():
        pe_ref[...] = row_ref[...][:, None, :] + col_ref[...][None, :, :]

    o_ref[...] = x_ref[...] + pe_ref[...][None]


def kernel(x, pos_row, pos_col):
    B, SEQ, D = x.shape
    x4 = x.reshape(B, _H, _W, D)
    out = pl.pallas_call(
        _add_body,
        grid=(B // _NB,),
        in_specs=[
            pl.BlockSpec((_NB, _H, _W, D), lambda b: (b, 0, 0, 0)),
            pl.BlockSpec((_H, D), lambda b: (0, 0)),
            pl.BlockSpec((_W, D), lambda b: (0, 0)),
        ],
        out_specs=pl.BlockSpec((_NB, _H, _W, D), lambda b: (b, 0, 0, 0)),
        out_shape=jax.ShapeDtypeStruct((B, _H, _W, D), x.dtype),
        scratch_shapes=[pltpu.VMEM((_H, _W, D), x.dtype)],
    )(x4, pos_row, pos_col)
    return out.reshape(B, SEQ, D)


# speedup vs baseline: 1.2641x; 1.0128x over previous
import jax
import jax.numpy as jnp
from jax.experimental import pallas as pl
from jax.experimental.pallas import tpu as pltpu

_H = 32
_W = 32
_NB = 4


def _add_body(x_ref, row_ref, col_ref, o_ref, pe_ref):
    @pl.when(pl.program_id(0) == 0)
    def _():
        pe_ref[...] = row_ref[...][:, None, :] + col_ref[...][None, :, :]

    o_ref[...] = x_ref[...] + pe_ref[...][None]


def kernel(x, pos_row, pos_col):
    B, SEQ, D = x.shape
    x4 = x.reshape(B, _H, _W, D)
    out = pl.pallas_call(
        _add_body,
        grid=(B // _NB,),
        in_specs=[
            pl.BlockSpec((_NB, _H, _W, D), lambda b: (b, 0, 0, 0)),
            pl.BlockSpec((_H, D), lambda b: (0, 0)),
            pl.BlockSpec((_W, D), lambda b: (0, 0)),
        ],
        out_specs=pl.BlockSpec((_NB, _H, _W, D), lambda b: (b, 0, 0, 0)),
        out_shape=jax.ShapeDtypeStruct((B, _H, _W, D), x.dtype),
        scratch_shapes=[pltpu.VMEM((_H, _W, D), x.dtype)],
    )(x4, pos_row, pos_col)
    return out.reshape(B, SEQ, D)
